# baseline (device time: 103540 ns/iter reference)
import jax
import jax.numpy as jnp
from jax import lax
from jax.experimental import pallas as pl
from jax.experimental.pallas import tpu as pltpu

N_DEV = 16

RING_SEQ = [0, 1, 5, 9, 13, 14, 10, 6, 2, 3, 7, 11, 15, 12, 8, 4]
RING_POS = [RING_SEQ.index(p) for p in range(N_DEV)]


def kernel(x, w_mat, scale_x, scale_w):
    m_per, k = x.shape
    _, n_per = w_mat.shape

    n_full = N_DEV // 2 - 1
    n_hops = n_full + 1
    n_sub = 2
    m_sub = m_per // n_sub

    my = lax.axis_index("i")
    pos = jnp.asarray(RING_POS, jnp.int32)[my]
    seq = jnp.asarray(RING_SEQ, jnp.int32)
    idx = jnp.arange(n_hops + 1, dtype=jnp.int32)
    r_slots = seq[(pos - idx) % N_DEV]
    l_slots = seq[(pos + idx) % N_DEV]
    meta = jnp.concatenate(
        [seq[jnp.stack([(pos + 1) % N_DEV, (pos - 1) % N_DEV])],
         r_slots, l_slots]
    )

    def body(x_ref, w_ref, sx_ref, sw_ref, meta_ref, out_ref,
             gather_ref,
             r_send_sems, r_recv_sems, l_send_sems, l_recv_sems):
        right = meta_ref[0]
        left = meta_ref[1]

        def r_slot(h):
            return meta_ref[2 + h]

        def l_slot(h):
            return meta_ref[11 + h]

        barrier_sem = pltpu.get_barrier_semaphore()
        for nbr in (left, right):
            pl.semaphore_signal(
                barrier_sem, inc=1,
                device_id=(nbr,), device_id_type=pl.DeviceIdType.MESH,
            )
        pl.semaphore_wait(barrier_sem, 2)

        scale = sx_ref[0, 0] * sw_ref[0, 0]

        def epilogue(rows, n_rows, acc):
            y = acc.astype(jnp.float32) * scale
            out_ref[pl.ds(rows, n_rows), :] = (
                y / (1.0 + jnp.exp(-jnp.clip(y, -60.0, 60.0)))
            )

        def compute_chunk(origin, chunk):
            acc = lax.dot_general(
                chunk, w_ref[...],
                (((1,), (0,)), ((), ())),
                preferred_element_type=jnp.int32,
            )
            epilogue(origin * m_per, m_per, acc)

        def compute(origin):
            compute_chunk(
                origin, gather_ref[pl.ds(origin, 1)].reshape(m_per, k)
            )

        def compute_sub(origin, s):
            chunk = gather_ref[
                pl.ds(origin, 1), pl.ds(s * m_sub, m_sub)
            ].reshape(m_sub, k)
            acc = lax.dot_general(
                chunk, w_ref[...],
                (((1,), (0,)), ((), ())),
                preferred_element_type=jnp.int32,
            )
            epilogue(origin * m_per + s * m_sub, m_sub, acc)

        def hop(h, s, slot, nbr, send_sems, recv_sems):
            rows = pl.ds(s * m_sub, m_sub)
            src = x_ref.at[rows] if h == 0 else gather_ref.at[slot, rows]
            return pltpu.make_async_remote_copy(
                src_ref=src,
                dst_ref=gather_ref.at[slot, rows],
                send_sem=send_sems.at[h, s],
                recv_sem=recv_sems.at[h, s],
                device_id=(nbr,),
                device_id_type=pl.DeviceIdType.MESH,
            )

        r_rdma = [[hop(h, s, r_slot(h), right, r_send_sems, r_recv_sems)
                   for s in range(n_sub if h < n_hops - 1 else 1)]
                  for h in range(n_hops)]
        l_rdma = [[hop(h, s, l_slot(h), left, l_send_sems, l_recv_sems)
                   for s in (range(n_sub) if h < n_hops - 1 else (1,))]
                  for h in range(n_hops)]

        for s in range(n_sub):
            r_rdma[0][s].start()
            l_rdma[0][s].start()
        compute_chunk(r_slot(0), x_ref[...])

        antipode = r_slot(8)
        for h in range(n_full):
            for s in range(n_sub):
                r_rdma[h][s].wait_recv()
                if h + 1 < n_full or s == 0:
                    r_rdma[h + 1][s if h + 1 < n_full else 0].start()
                l_rdma[h][s].wait_recv()
                if h + 1 < n_full or s == 1:
                    l_rdma[h + 1][s if h + 1 < n_full else 0].start()
            compute(r_slot(h + 1))
            compute(l_slot(h + 1))

        r_rdma[n_hops - 1][0].wait_recv()
        compute_sub(antipode, 0)
        l_rdma[n_hops - 1][0].wait_recv()
        compute_sub(antipode, 1)

        for hops in (r_rdma, l_rdma):
            for subs in hops:
                for r in subs:
                    r.wait_send()

    return pl.pallas_call(
        body,
        out_shape=jax.ShapeDtypeStruct((N_DEV * m_per, n_per), jnp.float32),
        in_specs=[
            pl.BlockSpec(memory_space=pltpu.VMEM),
            pl.BlockSpec(memory_space=pltpu.VMEM),
            pl.BlockSpec(memory_space=pltpu.SMEM),
            pl.BlockSpec(memory_space=pltpu.SMEM),
            pl.BlockSpec(memory_space=pltpu.SMEM),
        ],
        out_specs=pl.BlockSpec(memory_space=pltpu.VMEM),
        scratch_shapes=[
            pltpu.VMEM((N_DEV, m_per, k), jnp.int8),
            pltpu.SemaphoreType.DMA((n_hops, n_sub)),
            pltpu.SemaphoreType.DMA((n_hops, n_sub)),
            pltpu.SemaphoreType.DMA((n_hops, n_sub)),
            pltpu.SemaphoreType.DMA((n_hops, n_sub)),
        ],
        compiler_params=pltpu.CompilerParams(collective_id=0),
    )(x, w_mat, scale_x.reshape(1, 1), scale_w.reshape(1, 1), meta)


# device time: 99486 ns/iter; 1.0407x vs baseline; 1.0407x over previous
import jax
import jax.numpy as jnp
from jax import lax
from jax.experimental import pallas as pl
from jax.experimental.pallas import tpu as pltpu

N_DEV = 16

RING_SEQ = [0, 1, 5, 9, 13, 14, 10, 6, 2, 3, 7, 11, 15, 12, 8, 4]
RING_POS = [RING_SEQ.index(p) for p in range(N_DEV)]


def kernel(x, w_mat, scale_x, scale_w):
    m_per, k = x.shape
    _, n_per = w_mat.shape

    n_full = N_DEV // 2 - 1
    n_hops = n_full + 1
    n_sub = 2
    m_sub = m_per // n_sub

    tables = jnp.asarray(RING_SEQ + RING_POS, jnp.int32)

    def body(x_ref, w_ref, sx_ref, sw_ref, tab_ref, out_ref,
             gather_ref,
             r_send_sems, r_recv_sems, l_send_sems, l_recv_sems):
        my = lax.axis_index("i")
        pos = tab_ref[16 + my]

        def seq_at(p):
            return tab_ref[lax.rem(p + 2 * N_DEV, N_DEV)]

        right = seq_at(pos + 1)
        left = seq_at(pos - 1)

        def r_slot(h):
            return seq_at(pos - h)

        def l_slot(h):
            return seq_at(pos + h)

        barrier_sem = pltpu.get_barrier_semaphore()
        for nbr in (left, right):
            pl.semaphore_signal(
                barrier_sem, inc=1,
                device_id=(nbr,), device_id_type=pl.DeviceIdType.MESH,
            )
        pl.semaphore_wait(barrier_sem, 2)

        scale = sx_ref[0, 0] * sw_ref[0, 0]

        def epilogue(rows, n_rows, acc):
            y = acc.astype(jnp.float32) * scale
            out_ref[pl.ds(rows, n_rows), :] = (
                y / (1.0 + jnp.exp(-jnp.clip(y, -60.0, 60.0)))
            )

        def compute_chunk(origin, chunk):
            acc = lax.dot_general(
                chunk, w_ref[...],
                (((1,), (0,)), ((), ())),
                preferred_element_type=jnp.int32,
            )
            epilogue(origin * m_per, m_per, acc)

        def compute(origin):
            compute_chunk(
                origin, gather_ref[pl.ds(origin, 1)].reshape(m_per, k)
            )

        def compute_sub(origin, s):
            chunk = gather_ref[
                pl.ds(origin, 1), pl.ds(s * m_sub, m_sub)
            ].reshape(m_sub, k)
            acc = lax.dot_general(
                chunk, w_ref[...],
                (((1,), (0,)), ((), ())),
                preferred_element_type=jnp.int32,
            )
            epilogue(origin * m_per + s * m_sub, m_sub, acc)

        def hop(h, s, slot, nbr, send_sems, recv_sems):
            rows = pl.ds(s * m_sub, m_sub)
            src = x_ref.at[rows] if h == 0 else gather_ref.at[slot, rows]
            return pltpu.make_async_remote_copy(
                src_ref=src,
                dst_ref=gather_ref.at[slot, rows],
                send_sem=send_sems.at[h, s],
                recv_sem=recv_sems.at[h, s],
                device_id=(nbr,),
                device_id_type=pl.DeviceIdType.MESH,
            )

        r_rdma = [[hop(h, s, r_slot(h), right, r_send_sems, r_recv_sems)
                   for s in range(n_sub if h < n_hops - 1 else 1)]
                  for h in range(n_hops)]
        l_rdma = [[hop(h, s, l_slot(h), left, l_send_sems, l_recv_sems)
                   for s in (range(n_sub) if h < n_hops - 1 else (1,))]
                  for h in range(n_hops)]

        for s in range(n_sub):
            r_rdma[0][s].start()
            l_rdma[0][s].start()
        compute_chunk(r_slot(0), x_ref[...])

        antipode = r_slot(8)
        for h in range(n_full):
            for s in range(n_sub):
                r_rdma[h][s].wait_recv()
                if h + 1 < n_full or s == 0:
                    r_rdma[h + 1][s if h + 1 < n_full else 0].start()
                l_rdma[h][s].wait_recv()
                if h + 1 < n_full or s == 1:
                    l_rdma[h + 1][s if h + 1 < n_full else 0].start()
            compute(r_slot(h + 1))
            compute(l_slot(h + 1))

        r_rdma[n_hops - 1][0].wait_recv()
        compute_sub(antipode, 0)
        l_rdma[n_hops - 1][0].wait_recv()
        compute_sub(antipode, 1)

        for hops in (r_rdma, l_rdma):
            for subs in hops:
                for r in subs:
                    r.wait_send()

    return pl.pallas_call(
        body,
        out_shape=jax.ShapeDtypeStruct((N_DEV * m_per, n_per), jnp.float32),
        in_specs=[
            pl.BlockSpec(memory_space=pltpu.VMEM),
            pl.BlockSpec(memory_space=pltpu.VMEM),
            pl.BlockSpec(memory_space=pltpu.SMEM),
            pl.BlockSpec(memory_space=pltpu.SMEM),
            pl.BlockSpec(memory_space=pltpu.SMEM),
        ],
        out_specs=pl.BlockSpec(memory_space=pltpu.VMEM),
        scratch_shapes=[
            pltpu.VMEM((N_DEV, m_per, k), jnp.int8),
            pltpu.SemaphoreType.DMA((n_hops, n_sub)),
            pltpu.SemaphoreType.DMA((n_hops, n_sub)),
            pltpu.SemaphoreType.DMA((n_hops, n_sub)),
            pltpu.SemaphoreType.DMA((n_hops, n_sub)),
        ],
        compiler_params=pltpu.CompilerParams(collective_id=0),
    )(x, w_mat, scale_x.reshape(1, 1), scale_w.reshape(1, 1), tables)
